# baseline (speedup 1.0000x reference)
import jax
import jax.numpy as jnp
import numpy as np
from jax import lax
from jax.experimental import pallas as pl
from jax.experimental.pallas import tpu as pltpu, tpu_sc as plsc

HTS = 50000
D = 128
BATCH = 4
T = 8192
NW = 32
TOK_PER_W = (BATCH * T) // NW
CHUNK = 64
NCHUNK = TOK_PER_W // CHUNK
NSET = 12
XROW = T + 8

P2 = 66049
P3 = 16974593
P4 = 67503105
WRAP_FIX = 17296


def _umod_hts(h):
    r = lax.rem(h, jnp.int32(HTS))
    r = r + jnp.where(r < 0, jnp.int32(HTS), jnp.int32(0))
    r = r + jnp.where(h < 0, jnp.int32(WRAP_FIX), jnp.int32(0))
    r = r - jnp.where(r >= jnp.int32(HTS), jnp.int32(HTS), jnp.int32(0))
    return r


def _body(xf, w3, w4, w5, out,
          xbuf, i3, i4, i5, acc,
          ei3, ei4, e3, e4, obuf,
          gsem, wsem, es):
    i32 = jnp.int32
    c = lax.axis_index("c")
    s = lax.axis_index("s")
    wid = s * i32(2) + c
    row = wid // i32(8)
    seg = wid - row * i32(8)
    xoff = row * i32(XROW) + seg * i32(TOK_PER_W)
    outrow = wid * i32(TOK_PER_W)
    third = jnp.float32(1.0 / 3.0)

    pltpu.sync_copy(xf.at[pl.ds(xoff, TOK_PER_W + 8)], xbuf)

    def slot_refs(sid):
        soff = sid * i32(CHUNK)
        return (acc.at[pl.ds(soff, CHUNK)],
                i3.at[pl.ds(soff, CHUNK)],
                i4.at[pl.ds(soff, CHUNK)],
                i5.at[pl.ds(soff, CHUNK)])

    def compute_idx(obase, sid):
        soff = sid * i32(CHUNK)

        @pl.loop(i32(0), i32(CHUNK), step=i32(16))
        def _grp(gg):
            o = obase + gg
            v0 = xbuf[pl.ds(o + i32(4), 16)]
            v1 = xbuf[pl.ds(o + i32(3), 16)]
            v2 = xbuf[pl.ds(o + i32(2), 16)]
            v3 = xbuf[pl.ds(o + i32(1), 16)]
            v4 = xbuf[pl.ds(o, 16)]
            h3 = v2 * jnp.int32(P2) + v1 * jnp.int32(257) + v0
            h4 = h3 + v3 * jnp.int32(P3)
            h5 = h4 + v4 * jnp.int32(P4)
            sl = pl.ds(soff + gg, 16)
            i3[sl] = _umod_hts(h3)
            i4[sl] = _umod_hts(h4)
            i5[sl] = _umod_hts(h5)

    def zero_slot(sid):
        soff = sid * i32(CHUNK)
        z = jnp.zeros((16,), jnp.float32)

        @plsc.parallel_loop(soff, soff + i32(CHUNK), i32(1))
        def _z(tt):
            for col in range(D // 16):
                acc[tt, pl.ds(col * 16, 16)] = z

    def fire_gathers(sid):
        a, s3, s4, s5 = slot_refs(sid)
        pltpu.async_copy(w3.at[s3], a, gsem.at[sid], add=True)
        pltpu.async_copy(w4.at[s4], a, gsem.at[sid], add=True)
        pltpu.async_copy(w5.at[s5], a, gsem.at[sid], add=True)

    def drain_gathers(sid):
        a, s3, s4, s5 = slot_refs(sid)
        pltpu.make_async_copy(w3.at[s3], a, gsem.at[sid]).wait()
        pltpu.make_async_copy(w4.at[s4], a, gsem.at[sid]).wait()
        pltpu.make_async_copy(w5.at[s5], a, gsem.at[sid]).wait()

    def scale_slot(sid):
        soff = sid * i32(CHUNK)

        @plsc.parallel_loop(soff, soff + i32(CHUNK), i32(1))
        def _sc(tt):
            for col in range(D // 16):
                csl = pl.ds(col * 16, 16)
                acc[tt, csl] = acc[tt, csl] * third

    def fire_wb(sid, obase):
        a = slot_refs(sid)[0]
        pltpu.async_copy(a, out.at[pl.ds(outrow + obase, CHUNK)],
                         wsem.at[sid])

    def drain_wb(sid, obase):
        a = slot_refs(sid)[0]
        pltpu.make_async_copy(
            a, out.at[pl.ds(outrow + obase, CHUNK)], wsem.at[sid]).wait()

    @pl.loop(i32(0), i32(NCHUNK + 2))
    def _ring(it):
        @pl.when(it >= i32(2))
        def _consume():
            cid = it - i32(2)
            sid = lax.rem(cid, i32(NSET))
            drain_gathers(sid)
            scale_slot(sid)
            fire_wb(sid, cid * i32(CHUNK))

        @pl.when(it < i32(NCHUNK))
        def _produce():
            sid = lax.rem(it, i32(NSET))

            @pl.when(it >= i32(NSET))
            def _reuse():
                drain_wb(sid, (it - i32(NSET)) * i32(CHUNK))

            zero_slot(sid)
            compute_idx(it * i32(CHUNK), sid)
            fire_gathers(sid)

            @pl.when(it == i32(0))
            def _edge_fire():
                ei3[pl.ds(0, 16)] = i3[pl.ds(0, 16)]
                ei4[pl.ds(0, 16)] = i4[pl.ds(0, 16)]
                pltpu.async_copy(w3.at[ei3], e3, es)
                pltpu.async_copy(w4.at[ei4], e4, es)

    @pl.loop(i32(max(NCHUNK - NSET, 0)), i32(NCHUNK))
    def _draintail(kk):
        drain_wb(lax.rem(kk, i32(NSET)), kk * i32(CHUNK))

    pltpu.make_async_copy(w3.at[ei3], e3, es).wait()
    pltpu.make_async_copy(w4.at[ei4], e4, es).wait()

    @pl.when(seg == i32(0))
    def _edge():
        z = jnp.zeros((16,), jnp.float32)
        for col in range(D // 16):
            csl = pl.ds(col * 16, 16)
            obuf[0, csl] = z
            obuf[1, csl] = z
            obuf[2, csl] = e3[2, csl] * third
            obuf[3, csl] = (e3[3, csl] + e4[3, csl]) * third
        pltpu.sync_copy(obuf, out.at[pl.ds(outrow, 4)])


@jax.jit
def _sc_embed(xflat, w3, w4, w5):
    mesh = plsc.VectorSubcoreMesh(core_axis_name="c", subcore_axis_name="s")
    f = pl.kernel(
        _body,
        out_type=jax.ShapeDtypeStruct((BATCH * T, D), jnp.float32),
        mesh=mesh,
        scratch_types=[
            pltpu.VMEM((TOK_PER_W + 8,), jnp.int32),
            pltpu.VMEM((NSET * CHUNK,), jnp.int32),
            pltpu.VMEM((NSET * CHUNK,), jnp.int32),
            pltpu.VMEM((NSET * CHUNK,), jnp.int32),
            pltpu.VMEM((NSET * CHUNK, D), jnp.float32),
            pltpu.VMEM((16,), jnp.int32),
            pltpu.VMEM((16,), jnp.int32),
            pltpu.VMEM((16, D), jnp.float32),
            pltpu.VMEM((16, D), jnp.float32),
            pltpu.VMEM((4, D), jnp.float32),
            pltpu.SemaphoreType.DMA((NSET,)),
            pltpu.SemaphoreType.DMA((NSET,)),
            pltpu.SemaphoreType.DMA,
        ],
    )
    return f(xflat, w3, w4, w5)


def kernel(x, W3, W4, W5):
    x32 = x.astype(jnp.int32)
    xpad = jnp.pad(x32, ((0, 0), (4, 4)))
    xflat = xpad.reshape(-1)
    out = _sc_embed(xflat, W3, W4, W5)
    return out.reshape(BATCH, T, D)


# speedup vs baseline: 1.0276x; 1.0276x over previous
import jax
import jax.numpy as jnp
import numpy as np
from jax import lax
from jax.experimental import pallas as pl
from jax.experimental.pallas import tpu as pltpu, tpu_sc as plsc

HTS = 50000
D = 128
BATCH = 4
T = 8192
NW = 32
TOK_PER_W = (BATCH * T) // NW
CHUNK = 128
NCHUNK = TOK_PER_W // CHUNK
NSET = 7
XROW = T + 8

P2 = 66049
P3 = 16974593
P4 = 67503105
WRAP_FIX = 17296


def _umod_hts(h):
    r = lax.rem(h, jnp.int32(HTS))
    r = r + jnp.where(r < 0, jnp.int32(HTS), jnp.int32(0))
    r = r + jnp.where(h < 0, jnp.int32(WRAP_FIX), jnp.int32(0))
    r = r - jnp.where(r >= jnp.int32(HTS), jnp.int32(HTS), jnp.int32(0))
    return r


def _body(xf, w3, w4, w5, out,
          xbuf, i3, i4, i5, acc,
          ei3, ei4, e3, e4, obuf,
          gsem, wsem, es):
    i32 = jnp.int32
    c = lax.axis_index("c")
    s = lax.axis_index("s")
    wid = s * i32(2) + c
    row = wid // i32(8)
    seg = wid - row * i32(8)
    xoff = row * i32(XROW) + seg * i32(TOK_PER_W)
    outrow = wid * i32(TOK_PER_W)
    third = jnp.float32(1.0 / 3.0)

    pltpu.sync_copy(xf.at[pl.ds(xoff, TOK_PER_W + 8)], xbuf)

    def slot_refs(sid):
        soff = sid * i32(CHUNK)
        return (acc.at[pl.ds(soff, CHUNK)],
                i3.at[pl.ds(soff, CHUNK)],
                i4.at[pl.ds(soff, CHUNK)],
                i5.at[pl.ds(soff, CHUNK)])

    def compute_idx(obase, sid):
        soff = sid * i32(CHUNK)

        @pl.loop(i32(0), i32(CHUNK), step=i32(16))
        def _grp(gg):
            o = obase + gg
            v0 = xbuf[pl.ds(o + i32(4), 16)]
            v1 = xbuf[pl.ds(o + i32(3), 16)]
            v2 = xbuf[pl.ds(o + i32(2), 16)]
            v3 = xbuf[pl.ds(o + i32(1), 16)]
            v4 = xbuf[pl.ds(o, 16)]
            h3 = v2 * jnp.int32(P2) + v1 * jnp.int32(257) + v0
            h4 = h3 + v3 * jnp.int32(P3)
            h5 = h4 + v4 * jnp.int32(P4)
            sl = pl.ds(soff + gg, 16)
            i3[sl] = _umod_hts(h3)
            i4[sl] = _umod_hts(h4)
            i5[sl] = _umod_hts(h5)

    def zero_slot(sid):
        soff = sid * i32(CHUNK)
        z = jnp.zeros((16,), jnp.float32)

        @plsc.parallel_loop(soff, soff + i32(CHUNK), i32(1))
        def _z(tt):
            for col in range(D // 16):
                acc[tt, pl.ds(col * 16, 16)] = z

    def fire_gathers(sid):
        a, s3, s4, s5 = slot_refs(sid)
        pltpu.async_copy(w3.at[s3], a, gsem.at[sid], add=True)
        pltpu.async_copy(w4.at[s4], a, gsem.at[sid], add=True)
        pltpu.async_copy(w5.at[s5], a, gsem.at[sid], add=True)

    def drain_gathers(sid):
        a, s3, s4, s5 = slot_refs(sid)
        pltpu.make_async_copy(w3.at[s3], a, gsem.at[sid]).wait()
        pltpu.make_async_copy(w4.at[s4], a, gsem.at[sid]).wait()
        pltpu.make_async_copy(w5.at[s5], a, gsem.at[sid]).wait()

    def scale_slot(sid):
        soff = sid * i32(CHUNK)

        @plsc.parallel_loop(soff, soff + i32(CHUNK), i32(1))
        def _sc(tt):
            for col in range(D // 16):
                csl = pl.ds(col * 16, 16)
                acc[tt, csl] = acc[tt, csl] * third

    def fire_wb(sid, obase):
        a = slot_refs(sid)[0]
        pltpu.async_copy(a, out.at[pl.ds(outrow + obase, CHUNK)],
                         wsem.at[sid])

    def drain_wb(sid, obase):
        a = slot_refs(sid)[0]
        pltpu.make_async_copy(
            a, out.at[pl.ds(outrow + obase, CHUNK)], wsem.at[sid]).wait()

    @pl.loop(i32(0), i32(NCHUNK + 2))
    def _ring(it):
        @pl.when(it >= i32(2))
        def _consume():
            cid = it - i32(2)
            sid = lax.rem(cid, i32(NSET))
            drain_gathers(sid)
            scale_slot(sid)
            fire_wb(sid, cid * i32(CHUNK))

        @pl.when(it < i32(NCHUNK))
        def _produce():
            sid = lax.rem(it, i32(NSET))

            @pl.when(it >= i32(NSET))
            def _reuse():
                drain_wb(sid, (it - i32(NSET)) * i32(CHUNK))

            zero_slot(sid)
            compute_idx(it * i32(CHUNK), sid)
            fire_gathers(sid)

            @pl.when(it == i32(0))
            def _edge_fire():
                ei3[pl.ds(0, 16)] = i3[pl.ds(0, 16)]
                ei4[pl.ds(0, 16)] = i4[pl.ds(0, 16)]
                pltpu.async_copy(w3.at[ei3], e3, es)
                pltpu.async_copy(w4.at[ei4], e4, es)

    @pl.loop(i32(max(NCHUNK - NSET, 0)), i32(NCHUNK))
    def _draintail(kk):
        drain_wb(lax.rem(kk, i32(NSET)), kk * i32(CHUNK))

    pltpu.make_async_copy(w3.at[ei3], e3, es).wait()
    pltpu.make_async_copy(w4.at[ei4], e4, es).wait()

    @pl.when(seg == i32(0))
    def _edge():
        z = jnp.zeros((16,), jnp.float32)
        for col in range(D // 16):
            csl = pl.ds(col * 16, 16)
            obuf[0, csl] = z
            obuf[1, csl] = z
            obuf[2, csl] = e3[2, csl] * third
            obuf[3, csl] = (e3[3, csl] + e4[3, csl]) * third
        pltpu.sync_copy(obuf, out.at[pl.ds(outrow, 4)])


@jax.jit
def _sc_embed(xflat, w3, w4, w5):
    mesh = plsc.VectorSubcoreMesh(core_axis_name="c", subcore_axis_name="s")
    f = pl.kernel(
        _body,
        out_type=jax.ShapeDtypeStruct((BATCH * T, D), jnp.float32),
        mesh=mesh,
        scratch_types=[
            pltpu.VMEM((TOK_PER_W + 8,), jnp.int32),
            pltpu.VMEM((NSET * CHUNK,), jnp.int32),
            pltpu.VMEM((NSET * CHUNK,), jnp.int32),
            pltpu.VMEM((NSET * CHUNK,), jnp.int32),
            pltpu.VMEM((NSET * CHUNK, D), jnp.float32),
            pltpu.VMEM((16,), jnp.int32),
            pltpu.VMEM((16,), jnp.int32),
            pltpu.VMEM((16, D), jnp.float32),
            pltpu.VMEM((16, D), jnp.float32),
            pltpu.VMEM((4, D), jnp.float32),
            pltpu.SemaphoreType.DMA((NSET,)),
            pltpu.SemaphoreType.DMA((NSET,)),
            pltpu.SemaphoreType.DMA,
        ],
    )
    return f(xflat, w3, w4, w5)


def kernel(x, W3, W4, W5):
    x32 = x.astype(jnp.int32)
    xpad = jnp.pad(x32, ((0, 0), (4, 4)))
    xflat = xpad.reshape(-1)
    out = _sc_embed(xflat, W3, W4, W5)
    return out.reshape(BATCH, T, D)
